# Initial kernel scaffold; baseline (speedup 1.0000x reference)
import jax
import jax.numpy as jnp
from jax.experimental import pallas as pl


def kernel(x, edge_index, batch, W1, b1, W2, b2, Wfc, bfc):
    raise NotImplementedError("write your pallas kernel here")


# speedup vs baseline: 12.9806x; 12.9806x over previous
import functools

import jax
import jax.numpy as jnp
from jax import lax
from jax.experimental import pallas as pl
from jax.experimental.pallas import tpu as pltpu
from jax.experimental.pallas import tpu_sc as plsc

_N = 10000
_D = 128
_G = 64
_C = 10
_E = 320000
_NC = 2
_NS = 16
_NW = _NC * _NS
_LANE = 128
_NCH = 79
_EPAD = _NW * _NCH * _LANE
_R = 10240
_RPT = _R // _NS


@functools.lru_cache(maxsize=None)
def _build_deg_kernel():
    mesh = plsc.VectorSubcoreMesh(core_axis_name="c", subcore_axis_name="s")
    return functools.partial(
        pl.kernel,
        mesh=mesh,
        out_type=jax.ShapeDtypeStruct((_NC, _R), jnp.float32),
        scratch_types=[
            pltpu.VMEM((_NCH, _LANE), jnp.int32),
            pltpu.VMEM((_LANE,), jnp.float32),
            pltpu.VMEM((_RPT,), jnp.float32),
            pltpu.VMEM_SHARED((_R,), jnp.float32),
        ],
    )(_deg_body)


def _deg_body(dst_hbm, out_hbm, dstv, onesv, zv, dega):
    cid = lax.axis_index("c")
    sid = lax.axis_index("s")
    wid = sid * _NC + cid
    pltpu.sync_copy(dst_hbm.at[wid], dstv)
    for j in range(_LANE // 16):
        onesv[pl.ds(j * 16, 16)] = jnp.ones((16,), jnp.float32)

    def _zero(i, c):
        zv[pl.ds(i * 16, 16)] = jnp.zeros((16,), jnp.float32)
        return c

    lax.fori_loop(0, _RPT // 16, _zero, 0)
    pltpu.sync_copy(zv, dega.at[pl.ds(sid * _RPT, _RPT)])
    plsc.subcore_barrier()

    def _chunk(j, c):
        pltpu.sync_copy(onesv, dega.at[dstv.at[j]], add=True)
        return c

    lax.fori_loop(0, _NCH, _chunk, 0)
    plsc.subcore_barrier()
    pltpu.sync_copy(dega.at[pl.ds(sid * _RPT, _RPT)], zv)
    pltpu.sync_copy(zv, out_hbm.at[cid, pl.ds(sid * _RPT, _RPT)])


@functools.lru_cache(maxsize=None)
def _build_msg_kernel():
    mesh = plsc.VectorSubcoreMesh(core_axis_name="c", subcore_axis_name="s")
    return functools.partial(
        pl.kernel,
        mesh=mesh,
        out_type=jax.ShapeDtypeStruct((_NC, _R, _D), jnp.float32),
        scratch_types=[
            pltpu.VMEM((_NCH, _LANE), jnp.int32),
            pltpu.VMEM((_NCH, _LANE), jnp.int32),
            pltpu.VMEM((_LANE, _D), jnp.float32),
            pltpu.VMEM_SHARED((_R, _D), jnp.float32),
            pltpu.SemaphoreType.DMA,
        ],
    )(_msg_body)


def _msg_body(src_hbm, dst_hbm, xw_hbm, out_hbm, srcv, dstv, rows, acc, sem):
    cid = lax.axis_index("c")
    sid = lax.axis_index("s")
    wid = sid * _NC + cid
    pltpu.sync_copy(src_hbm.at[wid], srcv)
    pltpu.sync_copy(dst_hbm.at[wid], dstv)

    def _zero(i, c):
        for j in range(_D // 16):
            rows[i, pl.ds(j * 16, 16)] = jnp.zeros((16,), jnp.float32)
        return c

    lax.fori_loop(0, _LANE, _zero, 0)
    for t in range(_RPT // _LANE):
        pltpu.sync_copy(rows, acc.at[pl.ds(sid * _RPT + t * _LANE, _LANE)])
    plsc.subcore_barrier()

    def _chunk(j, c):
        pltpu.async_copy(xw_hbm.at[srcv.at[j]], rows, sem).wait()
        pltpu.sync_copy(rows, acc.at[dstv.at[j]], add=True)
        return c

    lax.fori_loop(0, _NCH, _chunk, 0)
    plsc.subcore_barrier()
    for t in range(_RPT // _LANE):
        sl = pl.ds(sid * _RPT + t * _LANE, _LANE)
        pltpu.sync_copy(acc.at[sl], rows)
        pltpu.sync_copy(rows, out_hbm.at[cid, sl])


def _scale_body(degt_ref, x_ref, w_ref, o_ref):
    deg = degt_ref[:, 0:1] + degt_ref[:, 1:2] + 1.0
    dinv = lax.rsqrt(deg)
    xw = jnp.dot(x_ref[...], w_ref[...], preferred_element_type=jnp.float32)
    o_ref[...] = xw * dinv


def _layer_body(a0_ref, a1_ref, xws_ref, degt_ref, b_ref, w_ref, o_ref):
    deg = degt_ref[:, 0:1] + degt_ref[:, 1:2] + 1.0
    dinv = lax.rsqrt(deg)
    h = (a0_ref[...] + a1_ref[...] + xws_ref[...]) * dinv + b_ref[...]
    h = jnp.maximum(h, 0.0)
    o_ref[...] = jnp.dot(h, w_ref[...], preferred_element_type=jnp.float32) * dinv


def _head_body(a0_ref, a1_ref, xws_ref, degt_ref, b_ref, batch_ref, wfc_ref,
               bfc_ref, o_ref):
    deg = degt_ref[:, 0:1] + degt_ref[:, 1:2] + 1.0
    dinv = lax.rsqrt(deg)
    h = (a0_ref[...] + a1_ref[...] + xws_ref[...]) * dinv + b_ref[...]
    h = jnp.maximum(h, 0.0)
    gids = lax.broadcasted_iota(jnp.int32, (_G, _N), 0)
    onehot = (batch_ref[...] == gids).astype(jnp.float32)
    sums = jnp.dot(onehot, h, preferred_element_type=jnp.float32)
    cnts = jnp.sum(onehot, axis=1, keepdims=True)
    g = sums / jnp.maximum(cnts, 1.0)
    logits = jnp.dot(g, wfc_ref[...], preferred_element_type=jnp.float32)
    logits = logits + bfc_ref[...]
    m = jnp.max(logits, axis=1, keepdims=True)
    s = logits - m
    o_ref[...] = s - jnp.log(jnp.sum(jnp.exp(s), axis=1, keepdims=True))


def _tc_call(body, out_shape, *args):
    return pl.pallas_call(
        body, out_shape=jax.ShapeDtypeStruct(out_shape, jnp.float32))(*args)


@jax.jit
def kernel(x, edge_index, batch, W1, b1, W2, b2, Wfc, bfc):
    src = edge_index[0].astype(jnp.int32)
    dst = edge_index[1].astype(jnp.int32)
    pad = _EPAD - _E
    srcp = jnp.concatenate([src, jnp.zeros((pad,), jnp.int32)])
    dstp = jnp.concatenate([dst, jnp.full((pad,), _N, jnp.int32)])
    srcp = srcp.reshape(_NW, _NCH, _LANE)
    dstp = dstp.reshape(_NW, _NCH, _LANE)

    deg_parts = _build_deg_kernel()(dstp)
    degt = jnp.transpose(deg_parts[:, :_N])

    xw1s = _tc_call(_scale_body, (_N, _D), degt, x, W1)
    acc1 = _build_msg_kernel()(srcp, dstp, xw1s)
    xw2s = _tc_call(_layer_body, (_N, _D),
                    acc1[0, :_N], acc1[1, :_N], xw1s, degt,
                    b1.reshape(1, _D), W2)
    acc2 = _build_msg_kernel()(srcp, dstp, xw2s)
    out = _tc_call(_head_body, (_G, _C),
                   acc2[0, :_N], acc2[1, :_N], xw2s, degt,
                   b2.reshape(1, _D), batch.astype(jnp.int32).reshape(1, _N),
                   Wfc, bfc.reshape(1, _C))
    return out
